# Initial kernel scaffold; baseline (speedup 1.0000x reference)
import jax
import jax.numpy as jnp
from jax.experimental import pallas as pl


def kernel(x, edge_index, W0m, b0m, W1m, b1m, W1v, b1v, noise):
    raise NotImplementedError("write your pallas kernel here")


# speedup vs baseline: 2.7737x; 2.7737x over previous
import functools

import jax
import jax.numpy as jnp
from jax import lax
from jax.experimental import pallas as pl
from jax.experimental.pallas import tpu as pltpu
from jax.experimental.pallas import tpu_sc as plsc

N = 10000
D = 128
E = 320000

_NT = 16
_CHUNK = 128
_CPT = 160
_G = 8
_NG = _CPT // _G
_EPT = _CPT * _CHUNK
_E_PAD = _NT * _EPT
_ACC_ROWS = 10240
_ZSTRIPE = _ACC_ROWS // _NT
_OSTRIPE = 640
_OLAST = N - 15 * _OSTRIPE


def _tc_layer0(x, W0m, b0m):
    def body(x_ref, w_ref, b_ref, mf_ref, vf_ref):
        h = jnp.dot(x_ref[...], w_ref[...],
                    preferred_element_type=jnp.float32) + b_ref[...]
        var = jnp.maximum(h, 0.0)
        mean = jnp.where(h > 0, h, jnp.exp(h) - 1.0)
        att = jnp.exp(-var)
        mf_ref[...] = mean * att
        vf_ref[...] = var * att * att

    return pl.pallas_call(
        body,
        grid=(5,),
        in_specs=[
            pl.BlockSpec((2000, D), lambda i: (i, 0)),
            pl.BlockSpec((D, D), lambda i: (0, 0)),
            pl.BlockSpec((1, D), lambda i: (0, 0)),
        ],
        out_specs=[pl.BlockSpec((2000, D), lambda i: (i, 0))] * 2,
        out_shape=[jax.ShapeDtypeStruct((N, D), jnp.float32)] * 2,
    )(x, W0m, b0m.reshape(1, D))


def _tc_layer1(mean_agg, var_agg, W1m, b1m, W1v, b1v):
    def body(m_ref, v_ref, wm_ref, bm_ref, wv_ref, bv_ref, mf_ref, vf_ref):
        m_in = m_ref[...]
        v_in = v_ref[...]
        m_in = jnp.where(m_in > 0, m_in, jnp.exp(m_in) - 1.0)
        v_in = jnp.where(v_in > 0, v_in, jnp.exp(v_in) - 1.0)
        mean = jnp.dot(m_in, wm_ref[...],
                       preferred_element_type=jnp.float32) + bm_ref[...]
        var = jnp.dot(v_in, wv_ref[...],
                      preferred_element_type=jnp.float32) + bv_ref[...]
        mean = jnp.where(mean > 0, mean, jnp.exp(mean) - 1.0)
        var = jnp.maximum(var, 0.0)
        att = jnp.exp(-var)
        mf_ref[...] = mean * att
        vf_ref[...] = var * att * att

    return pl.pallas_call(
        body,
        grid=(5,),
        in_specs=[
            pl.BlockSpec((2000, D), lambda i: (i, 0)),
            pl.BlockSpec((2000, D), lambda i: (i, 0)),
            pl.BlockSpec((D, D), lambda i: (0, 0)),
            pl.BlockSpec((1, D), lambda i: (0, 0)),
            pl.BlockSpec((D, D), lambda i: (0, 0)),
            pl.BlockSpec((1, D), lambda i: (0, 0)),
        ],
        out_specs=[pl.BlockSpec((2000, D), lambda i: (i, 0))] * 2,
        out_shape=[jax.ShapeDtypeStruct((N, D), jnp.float32)] * 2,
    )(mean_agg, var_agg, W1m, b1m.reshape(1, D), W1v, b1v.reshape(1, D))


def _tc_sample(mean2, var2, noise):
    def body(m_ref, v_ref, n_ref, o_ref):
        o_ref[...] = m_ref[...] + n_ref[...] * jnp.sqrt(v_ref[...])

    return pl.pallas_call(
        body,
        grid=(5,),
        in_specs=[pl.BlockSpec((2000, D), lambda i: (i, 0))] * 3,
        out_specs=pl.BlockSpec((2000, D), lambda i: (i, 0)),
        out_shape=jax.ShapeDtypeStruct((N, D), jnp.float32),
    )(mean2, var2, noise)


def _sc_spmm(mean_tab, var_tab, src3, dst3, zeros_tile):
    mesh = plsc.VectorSubcoreMesh(core_axis_name="c", subcore_axis_name="s")

    @functools.partial(
        pl.kernel,
        out_type=[jax.ShapeDtypeStruct((N, D), jnp.float32)] * 2,
        mesh=mesh,
        scratch_types=[
            pltpu.VMEM((_G, _CHUNK), jnp.int32),
            pltpu.VMEM((_G, _CHUNK), jnp.int32),
            pltpu.VMEM((_CHUNK, D), jnp.float32),
            pltpu.VMEM_SHARED((_ACC_ROWS, D), jnp.float32),
            pltpu.SemaphoreType.DMA,
        ],
    )
    def k(mean_h, var_h, src_h, dst_h, zer_h,
          mout, vout, sidx, didx, rows, acc, sem):
        c = lax.axis_index("c")
        s = lax.axis_index("s")

        pltpu.sync_copy(zer_h, rows)
        for j in range(_ZSTRIPE // _CHUNK):
            pltpu.sync_copy(rows, acc.at[pl.ds(s * _ZSTRIPE + j * _CHUNK,
                                               _CHUNK)])
        plsc.subcore_barrier()

        def make_body(tab):
            def body(g, carry):
                pltpu.sync_copy(src_h.at[s, pl.ds(g * _G, _G)], sidx)
                pltpu.sync_copy(dst_h.at[s, pl.ds(g * _G, _G)], didx)
                for j in range(_G):
                    pltpu.async_copy(tab.at[sidx.at[j]], rows, sem).wait()
                    pltpu.sync_copy(rows, acc.at[didx.at[j]], add=True)
                return carry
            return body

        @pl.when(c == 0)
        def _():
            lax.fori_loop(0, _NG, make_body(mean_h), 0)

        @pl.when(c == 1)
        def _():
            lax.fori_loop(0, _NG, make_body(var_h), 0)

        plsc.subcore_barrier()

        def copy_out(out_h):
            @pl.when(s < 15)
            def _():
                pltpu.sync_copy(acc.at[pl.ds(s * _OSTRIPE, _OSTRIPE)],
                                out_h.at[pl.ds(s * _OSTRIPE, _OSTRIPE)])

            @pl.when(s == 15)
            def _():
                pltpu.sync_copy(acc.at[pl.ds(15 * _OSTRIPE, _OLAST)],
                                out_h.at[pl.ds(15 * _OSTRIPE, _OLAST)])

        @pl.when(c == 0)
        def _():
            copy_out(mout)

        @pl.when(c == 1)
        def _():
            copy_out(vout)

    return k(mean_tab, var_tab, src3, dst3, zeros_tile)


def kernel(x, edge_index, W0m, b0m, W1m, b1m, W1v, b1v, noise):
    dst = edge_index[0]
    src = edge_index[1]
    pad = _E_PAD - E
    src3 = jnp.pad(src, (0, pad)).reshape(_NT, _CPT, _CHUNK)
    dst3 = jnp.pad(dst, (0, pad), constant_values=N).reshape(_NT, _CPT, _CHUNK)
    zeros_tile = jnp.zeros((_CHUNK, D), jnp.float32)

    mf0, vf0 = _tc_layer0(x, W0m, b0m)
    m_agg, v_agg = _sc_spmm(mf0, vf0, src3, dst3, zeros_tile)
    mf1, vf1 = _tc_layer1(m_agg, v_agg, W1m, b1m, W1v, b1v)
    m2, v2 = _sc_spmm(mf1, vf1, src3, dst3, zeros_tile)
    return _tc_sample(m2, v2, noise)


# speedup vs baseline: 2.9954x; 1.0799x over previous
import functools

import jax
import jax.numpy as jnp
from jax import lax
from jax.experimental import pallas as pl
from jax.experimental.pallas import tpu as pltpu
from jax.experimental.pallas import tpu_sc as plsc

N = 10000
D = 128
E = 320000

_NT = 16
_CHUNK = 128
_CPT = 160
_G = 8
_NG = _CPT // _G
_EPT = _CPT * _CHUNK
_E_PAD = _NT * _EPT
_ACC_ROWS = 10240
_ZSTRIPE = _ACC_ROWS // _NT
_OSTRIPE = 640
_OLAST = N - 15 * _OSTRIPE


def _tc_layer0(x, W0m, b0m):
    def body(x_ref, w_ref, b_ref, mf_ref, vf_ref):
        h = jnp.dot(x_ref[...], w_ref[...],
                    preferred_element_type=jnp.float32) + b_ref[...]
        var = jnp.maximum(h, 0.0)
        mean = jnp.where(h > 0, h, jnp.exp(h) - 1.0)
        att = jnp.exp(-var)
        mf_ref[...] = mean * att
        vf_ref[...] = var * att * att

    return pl.pallas_call(
        body,
        grid=(5,),
        in_specs=[
            pl.BlockSpec((2000, D), lambda i: (i, 0)),
            pl.BlockSpec((D, D), lambda i: (0, 0)),
            pl.BlockSpec((1, D), lambda i: (0, 0)),
        ],
        out_specs=[pl.BlockSpec((2000, D), lambda i: (i, 0))] * 2,
        out_shape=[jax.ShapeDtypeStruct((N, D), jnp.float32)] * 2,
    )(x, W0m, b0m.reshape(1, D))


def _tc_layer1(mean_agg, var_agg, W1m, b1m, W1v, b1v):
    def body(m_ref, v_ref, wm_ref, bm_ref, wv_ref, bv_ref, mf_ref, vf_ref):
        m_in = m_ref[...]
        v_in = v_ref[...]
        m_in = jnp.where(m_in > 0, m_in, jnp.exp(m_in) - 1.0)
        v_in = jnp.where(v_in > 0, v_in, jnp.exp(v_in) - 1.0)
        mean = jnp.dot(m_in, wm_ref[...],
                       preferred_element_type=jnp.float32) + bm_ref[...]
        var = jnp.dot(v_in, wv_ref[...],
                      preferred_element_type=jnp.float32) + bv_ref[...]
        mean = jnp.where(mean > 0, mean, jnp.exp(mean) - 1.0)
        var = jnp.maximum(var, 0.0)
        att = jnp.exp(-var)
        mf_ref[...] = mean * att
        vf_ref[...] = var * att * att

    return pl.pallas_call(
        body,
        grid=(5,),
        in_specs=[
            pl.BlockSpec((2000, D), lambda i: (i, 0)),
            pl.BlockSpec((2000, D), lambda i: (i, 0)),
            pl.BlockSpec((D, D), lambda i: (0, 0)),
            pl.BlockSpec((1, D), lambda i: (0, 0)),
            pl.BlockSpec((D, D), lambda i: (0, 0)),
            pl.BlockSpec((1, D), lambda i: (0, 0)),
        ],
        out_specs=[pl.BlockSpec((2000, D), lambda i: (i, 0))] * 2,
        out_shape=[jax.ShapeDtypeStruct((N, D), jnp.float32)] * 2,
    )(mean_agg, var_agg, W1m, b1m.reshape(1, D), W1v, b1v.reshape(1, D))


def _tc_sample(mean2, var2, noise):
    def body(m_ref, v_ref, n_ref, o_ref):
        o_ref[...] = m_ref[...] + n_ref[...] * jnp.sqrt(v_ref[...])

    return pl.pallas_call(
        body,
        grid=(5,),
        in_specs=[pl.BlockSpec((2000, D), lambda i: (i, 0))] * 3,
        out_specs=pl.BlockSpec((2000, D), lambda i: (i, 0)),
        out_shape=jax.ShapeDtypeStruct((N, D), jnp.float32),
    )(mean2, var2, noise)


def _sc_spmm(mean_tab, var_tab, src3, dst3, zeros_tile):
    mesh = plsc.VectorSubcoreMesh(core_axis_name="c", subcore_axis_name="s")

    @functools.partial(
        pl.kernel,
        out_type=[jax.ShapeDtypeStruct((N, D), jnp.float32)] * 2,
        mesh=mesh,
        scratch_types=[
            pltpu.VMEM((_G, _CHUNK), jnp.int32),
            pltpu.VMEM((_G, _CHUNK), jnp.int32),
            pltpu.VMEM((_CHUNK, D), jnp.float32),
            pltpu.VMEM((_CHUNK, D), jnp.float32),
            pltpu.VMEM_SHARED((_ACC_ROWS, D), jnp.float32),
            pltpu.SemaphoreType.DMA,
            pltpu.SemaphoreType.DMA,
            pltpu.SemaphoreType.DMA,
            pltpu.SemaphoreType.DMA,
        ],
    )
    def k(mean_h, var_h, src_h, dst_h, zer_h,
          mout, vout, sidx, didx, rows_a, rows_b, acc, ga, gb, sa, sb):
        c = lax.axis_index("c")
        s = lax.axis_index("s")

        pltpu.sync_copy(zer_h, rows_a)
        for j in range(_ZSTRIPE // _CHUNK):
            pltpu.sync_copy(rows_a, acc.at[pl.ds(s * _ZSTRIPE + j * _CHUNK,
                                                 _CHUNK)])
        plsc.subcore_barrier()

        rows = (rows_a, rows_b)
        gsem = (ga, gb)
        ssem = (sa, sb)

        def make_body(tab):
            def body(g, carry):
                pltpu.sync_copy(src_h.at[s, pl.ds(g * _G, _G)], sidx)
                pltpu.sync_copy(dst_h.at[s, pl.ds(g * _G, _G)], didx)
                gd = [None, None]
                sd = [None, None]
                gd[0] = pltpu.async_copy(tab.at[sidx.at[0]], rows_a, ga)
                for j in range(_G):
                    cur = j & 1
                    gd[cur].wait()
                    if j < _G - 1:
                        nxt = 1 - cur
                        if sd[nxt] is not None:
                            sd[nxt].wait()
                        gd[nxt] = pltpu.async_copy(
                            tab.at[sidx.at[j + 1]], rows[nxt], gsem[nxt])
                    sd[cur] = pltpu.async_copy(
                        rows[cur], acc.at[didx.at[j]], ssem[cur], add=True)
                sd[0].wait()
                sd[1].wait()
                return carry
            return body

        @pl.when(c == 0)
        def _():
            lax.fori_loop(0, _NG, make_body(mean_h), 0)

        @pl.when(c == 1)
        def _():
            lax.fori_loop(0, _NG, make_body(var_h), 0)

        plsc.subcore_barrier()

        def copy_out(out_h):
            @pl.when(s < 15)
            def _():
                pltpu.sync_copy(acc.at[pl.ds(s * _OSTRIPE, _OSTRIPE)],
                                out_h.at[pl.ds(s * _OSTRIPE, _OSTRIPE)])

            @pl.when(s == 15)
            def _():
                pltpu.sync_copy(acc.at[pl.ds(15 * _OSTRIPE, _OLAST)],
                                out_h.at[pl.ds(15 * _OSTRIPE, _OLAST)])

        @pl.when(c == 0)
        def _():
            copy_out(mout)

        @pl.when(c == 1)
        def _():
            copy_out(vout)

    return k(mean_tab, var_tab, src3, dst3, zeros_tile)


def kernel(x, edge_index, W0m, b0m, W1m, b1m, W1v, b1v, noise):
    dst = edge_index[0]
    src = edge_index[1]
    pad = _E_PAD - E
    src3 = jnp.pad(src, (0, pad)).reshape(_NT, _CPT, _CHUNK)
    dst3 = jnp.pad(dst, (0, pad), constant_values=N).reshape(_NT, _CPT, _CHUNK)
    zeros_tile = jnp.zeros((_CHUNK, D), jnp.float32)

    mf0, vf0 = _tc_layer0(x, W0m, b0m)
    m_agg, v_agg = _sc_spmm(mf0, vf0, src3, dst3, zeros_tile)
    mf1, vf1 = _tc_layer1(m_agg, v_agg, W1m, b1m, W1v, b1v)
    m2, v2 = _sc_spmm(mf1, vf1, src3, dst3, zeros_tile)
    return _tc_sample(m2, v2, noise)


# speedup vs baseline: 3.2526x; 1.0859x over previous
import functools

import jax
import jax.numpy as jnp
from jax import lax
from jax.experimental import pallas as pl
from jax.experimental.pallas import tpu as pltpu
from jax.experimental.pallas import tpu_sc as plsc

N = 10000
D = 128
E = 320000

_NT = 16
_CHUNK = 128
_CPT = 160
_G = 8
_NG = _CPT // _G
_EPT = _CPT * _CHUNK
_E_PAD = _NT * _EPT
_ACC_ROWS = 10240
_ZSTRIPE = _ACC_ROWS // _NT
_OSTRIPE = 640
_OLAST = N - 15 * _OSTRIPE


def _tc_layer0(x, W0m, b0m):
    def body(x_ref, w_ref, b_ref, mf_ref, vf_ref):
        h = jnp.dot(x_ref[...], w_ref[...],
                    preferred_element_type=jnp.float32) + b_ref[...]
        var = jnp.maximum(h, 0.0)
        mean = jnp.where(h > 0, h, jnp.exp(h) - 1.0)
        att = jnp.exp(-var)
        mf_ref[...] = mean * att
        vf_ref[...] = var * att * att

    return pl.pallas_call(
        body,
        grid=(5,),
        in_specs=[
            pl.BlockSpec((2000, D), lambda i: (i, 0)),
            pl.BlockSpec((D, D), lambda i: (0, 0)),
            pl.BlockSpec((1, D), lambda i: (0, 0)),
        ],
        out_specs=[pl.BlockSpec((2000, D), lambda i: (i, 0))] * 2,
        out_shape=[jax.ShapeDtypeStruct((N, D), jnp.float32)] * 2,
    )(x, W0m, b0m.reshape(1, D))


def _tc_layer1(mean_agg, var_agg, W1m, b1m, W1v, b1v):
    def body(m_ref, v_ref, wm_ref, bm_ref, wv_ref, bv_ref, mf_ref, vf_ref):
        m_in = m_ref[...]
        v_in = v_ref[...]
        m_in = jnp.where(m_in > 0, m_in, jnp.exp(m_in) - 1.0)
        v_in = jnp.where(v_in > 0, v_in, jnp.exp(v_in) - 1.0)
        mean = jnp.dot(m_in, wm_ref[...],
                       preferred_element_type=jnp.float32) + bm_ref[...]
        var = jnp.dot(v_in, wv_ref[...],
                      preferred_element_type=jnp.float32) + bv_ref[...]
        mean = jnp.where(mean > 0, mean, jnp.exp(mean) - 1.0)
        var = jnp.maximum(var, 0.0)
        att = jnp.exp(-var)
        mf_ref[...] = mean * att
        vf_ref[...] = var * att * att

    return pl.pallas_call(
        body,
        grid=(5,),
        in_specs=[
            pl.BlockSpec((2000, D), lambda i: (i, 0)),
            pl.BlockSpec((2000, D), lambda i: (i, 0)),
            pl.BlockSpec((D, D), lambda i: (0, 0)),
            pl.BlockSpec((1, D), lambda i: (0, 0)),
            pl.BlockSpec((D, D), lambda i: (0, 0)),
            pl.BlockSpec((1, D), lambda i: (0, 0)),
        ],
        out_specs=[pl.BlockSpec((2000, D), lambda i: (i, 0))] * 2,
        out_shape=[jax.ShapeDtypeStruct((N, D), jnp.float32)] * 2,
    )(mean_agg, var_agg, W1m, b1m.reshape(1, D), W1v, b1v.reshape(1, D))


def _tc_sample(mean2, var2, noise):
    def body(m_ref, v_ref, n_ref, o_ref):
        o_ref[...] = m_ref[...] + n_ref[...] * jnp.sqrt(v_ref[...])

    return pl.pallas_call(
        body,
        grid=(5,),
        in_specs=[pl.BlockSpec((2000, D), lambda i: (i, 0))] * 3,
        out_specs=pl.BlockSpec((2000, D), lambda i: (i, 0)),
        out_shape=jax.ShapeDtypeStruct((N, D), jnp.float32),
    )(mean2, var2, noise)


def _sc_spmm(mean_tab, var_tab, src3, dst3, zeros_tile):
    mesh = plsc.VectorSubcoreMesh(core_axis_name="c", subcore_axis_name="s")

    @functools.partial(
        pl.kernel,
        out_type=[jax.ShapeDtypeStruct((N, D), jnp.float32)] * 2,
        mesh=mesh,
        scratch_types=[
            pltpu.VMEM((_G, _CHUNK), jnp.int32),
            pltpu.VMEM((_G, _CHUNK), jnp.int32),
            pltpu.VMEM((_CHUNK, D), jnp.float32),
            pltpu.VMEM((_CHUNK, D), jnp.float32),
            pltpu.VMEM_SHARED((_ACC_ROWS, D), jnp.float32),
            pltpu.SemaphoreType.DMA,
            pltpu.SemaphoreType.DMA,
            pltpu.SemaphoreType.DMA,
            pltpu.SemaphoreType.DMA,
        ],
    )
    def k(mean_h, var_h, src_h, dst_h, zer_h,
          mout, vout, sidx, didx, rows_a, rows_b, acc, ga, gb, sa, sb):
        c = lax.axis_index("c")
        s = lax.axis_index("s")

        pltpu.sync_copy(zer_h, rows_a)
        for j in range(_ZSTRIPE // _CHUNK):
            pltpu.sync_copy(rows_a, acc.at[pl.ds(s * _ZSTRIPE + j * _CHUNK,
                                                 _CHUNK)])
        plsc.subcore_barrier()

        rows = (rows_a, rows_b)
        gsem = (ga, gb)
        ssem = (sa, sb)

        def make_body(tab):
            def body(g, carry):
                pltpu.sync_copy(src_h.at[s, pl.ds(g * _G, _G)], sidx)
                pltpu.sync_copy(dst_h.at[s, pl.ds(g * _G, _G)], didx)
                gd = [None, None]
                sd = [None, None]
                gd[0] = pltpu.async_copy(tab.at[sidx.at[0]], rows_a, ga)
                for j in range(_G):
                    cur = j & 1
                    if j < _G - 1:
                        nxt = 1 - cur
                        if sd[nxt] is not None:
                            sd[nxt].wait()
                        gd[nxt] = pltpu.async_copy(
                            tab.at[sidx.at[j + 1]], rows[nxt], gsem[nxt])
                    gd[cur].wait()
                    sd[cur] = pltpu.async_copy(
                        rows[cur], acc.at[didx.at[j]], ssem[cur], add=True)
                sd[0].wait()
                sd[1].wait()
                return carry
            return body

        @pl.when(c == 0)
        def _():
            lax.fori_loop(0, _NG, make_body(mean_h), 0)

        @pl.when(c == 1)
        def _():
            lax.fori_loop(0, _NG, make_body(var_h), 0)

        plsc.subcore_barrier()

        def copy_out(out_h):
            @pl.when(s < 15)
            def _():
                pltpu.sync_copy(acc.at[pl.ds(s * _OSTRIPE, _OSTRIPE)],
                                out_h.at[pl.ds(s * _OSTRIPE, _OSTRIPE)])

            @pl.when(s == 15)
            def _():
                pltpu.sync_copy(acc.at[pl.ds(15 * _OSTRIPE, _OLAST)],
                                out_h.at[pl.ds(15 * _OSTRIPE, _OLAST)])

        @pl.when(c == 0)
        def _():
            copy_out(mout)

        @pl.when(c == 1)
        def _():
            copy_out(vout)

    return k(mean_tab, var_tab, src3, dst3, zeros_tile)


def kernel(x, edge_index, W0m, b0m, W1m, b1m, W1v, b1v, noise):
    dst = edge_index[0]
    src = edge_index[1]
    pad = _E_PAD - E
    src3 = jnp.pad(src, (0, pad)).reshape(_NT, _CPT, _CHUNK)
    dst3 = jnp.pad(dst, (0, pad), constant_values=N).reshape(_NT, _CPT, _CHUNK)
    zeros_tile = jnp.zeros((_CHUNK, D), jnp.float32)

    mf0, vf0 = _tc_layer0(x, W0m, b0m)
    m_agg, v_agg = _sc_spmm(mf0, vf0, src3, dst3, zeros_tile)
    mf1, vf1 = _tc_layer1(m_agg, v_agg, W1m, b1m, W1v, b1v)
    m2, v2 = _sc_spmm(mf1, vf1, src3, dst3, zeros_tile)
    return _tc_sample(m2, v2, noise)
